# baseline (speedup 1.0000x reference)
import functools

import jax
import jax.numpy as jnp
from jax import lax
from jax.experimental import pallas as pl
from jax.experimental.pallas import tpu as pltpu
from jax.experimental.pallas import tpu_sc as plsc

_NUM_FIELDS = 26
_VOCAB = 100000
_EMBED_DIM = 32
_BATCH = 4096


@functools.cache
def _make_kernel():
    info = plsc.get_sparse_core_info()
    nc, ns, lanes = info.num_cores, info.num_subcores, info.num_lanes
    nw = nc * ns
    total = _BATCH * _NUM_FIELDS
    per_w = total // nw
    chunk = 128
    n_chunks = per_w // chunk
    vecs = chunk // lanes

    mesh = plsc.VectorSubcoreMesh(core_axis_name="c", subcore_axis_name="s")

    @functools.partial(
        pl.kernel,
        mesh=mesh,
        compiler_params=pltpu.CompilerParams(use_tc_tiling_on_sc=False),
        out_type=jax.ShapeDtypeStruct((total, _EMBED_DIM), jnp.float32),
        scratch_types=[
            pltpu.VMEM((n_chunks, chunk), jnp.int32),
            pltpu.VMEM((per_w, _EMBED_DIM), jnp.float32),
            pltpu.SemaphoreType.DMA,
        ],
    )
    def emb_lookup(idx_hbm, tab_hbm, out_hbm, idx_v, rows_v, sem):
        wid = lax.axis_index("s") * nc + lax.axis_index("c")
        base = wid * per_w

        pltpu.sync_copy(idx_hbm.at[wid], idx_v)

        lane = lax.iota(jnp.int32, lanes)

        def issue(j, carry):
            for t in range(vecs):
                pos = base + j * chunk + t * lanes + lane
                row = idx_v[j, pl.ds(t * lanes, lanes)] + (pos % _NUM_FIELDS) * _VOCAB
                idx_v[j, pl.ds(t * lanes, lanes)] = row
            pltpu.async_copy(
                tab_hbm.at[idx_v.at[j]],
                rows_v.at[pl.ds(j * chunk, chunk)],
                sem,
            )
            return carry

        lax.fori_loop(0, n_chunks, issue, 0)

        pltpu.make_async_copy(out_hbm.at[pl.ds(0, per_w)], rows_v, sem).wait()

        pltpu.sync_copy(rows_v, out_hbm.at[pl.ds(base, per_w)])

    return emb_lookup, nw, n_chunks, chunk


def kernel(indices, tables):
    emb_lookup, nw, n_chunks, chunk = _make_kernel()
    flat_tab = tables.reshape(_NUM_FIELDS * _VOCAB, _EMBED_DIM)
    flat_idx = indices.astype(jnp.int32).reshape(nw, n_chunks, chunk)
    out = emb_lookup(flat_idx, flat_tab)
    return out.reshape(_BATCH, _NUM_FIELDS, _EMBED_DIM)


# speedup vs baseline: 1.9288x; 1.9288x over previous
import functools

import jax
import jax.numpy as jnp
from jax import lax
from jax.experimental import pallas as pl
from jax.experimental.pallas import tpu as pltpu
from jax.experimental.pallas import tpu_sc as plsc

_NUM_FIELDS = 26
_VOCAB = 100000
_EMBED_DIM = 32
_BATCH = 4096


@functools.cache
def _make_kernel():
    info = plsc.get_sparse_core_info()
    nc, ns, lanes = info.num_cores, info.num_subcores, info.num_lanes
    nw = nc * ns
    n_rows = _NUM_FIELDS * _EMBED_DIM
    rows_pw = n_rows // nw
    chunk = 128
    n_chunks = _BATCH // chunk
    vecs = chunk // lanes

    mesh = plsc.VectorSubcoreMesh(core_axis_name="c", subcore_axis_name="s")

    @functools.partial(
        pl.kernel,
        mesh=mesh,
        compiler_params=pltpu.CompilerParams(use_tc_tiling_on_sc=False),
        out_type=jax.ShapeDtypeStruct((n_rows, _BATCH), jnp.float32),
        scratch_types=[
            pltpu.VMEM((2, n_chunks, chunk), jnp.int32),
            pltpu.VMEM((2, _BATCH), jnp.float32),
            pltpu.SemaphoreType.DMA,
            pltpu.SemaphoreType.DMA,
        ],
    )
    def emb_lookup(idx_hbm, tab_hbm, out_hbm, idx_v, rows_v, idx_sem, gat_sem):
        wid = lax.axis_index("s") * nc + lax.axis_index("c")
        r0 = wid * rows_pw

        pltpu.async_copy(idx_hbm.at[r0 // _EMBED_DIM], idx_v.at[0], idx_sem)

        def row_step(k, carry):
            r = r0 + k
            f = r // _EMBED_DIM
            base = r * _VOCAB

            pltpu.make_async_copy(idx_hbm.at[0], idx_v.at[k % 2],
                                  idx_sem).wait()

            @pl.when(k + 1 < rows_pw)
            def _prefetch_idx():
                fn = (r + 1) // _EMBED_DIM
                pltpu.async_copy(idx_hbm.at[fn], idx_v.at[(k + 1) % 2],
                                 idx_sem)

            for j in range(n_chunks):
                for t in range(vecs):
                    sl = (k % 2, j, pl.ds(t * lanes, lanes))
                    idx_v[sl] = idx_v[sl] + base
                pltpu.async_copy(
                    tab_hbm.at[idx_v.at[k % 2, j]],
                    rows_v.at[k % 2, pl.ds(j * chunk, chunk)],
                    gat_sem,
                )

            pltpu.make_async_copy(out_hbm.at[0], rows_v.at[k % 2],
                                  gat_sem).wait()
            pltpu.sync_copy(rows_v.at[k % 2], out_hbm.at[r])

            return carry

        lax.fori_loop(0, rows_pw, row_step, 0)

    return emb_lookup, n_chunks, chunk


def kernel(indices, tables):
    emb_lookup, n_chunks, chunk = _make_kernel()
    idx_t = indices.astype(jnp.int32).T.reshape(_NUM_FIELDS, n_chunks, chunk)
    flat_tab = tables.transpose(0, 2, 1).reshape(-1)
    out = emb_lookup(idx_t, flat_tab)
    return out.reshape(_NUM_FIELDS, _EMBED_DIM, _BATCH).transpose(2, 0, 1)
